# Initial kernel scaffold; baseline (speedup 1.0000x reference)
import jax
import jax.numpy as jnp
from jax.experimental import pallas as pl


def kernel(x, edge_index, edge_attr, batch, params):
    raise NotImplementedError("write your pallas kernel here")


# speedup vs baseline: 1.9564x; 1.9564x over previous
import functools

import jax
import jax.numpy as jnp
from jax import lax
from jax.experimental import pallas as pl
from jax.experimental.pallas import tpu as pltpu
from jax.experimental.pallas import tpu_sc as plsc

N = 50000
E = 800000
G = 1024
H = 200
HP = 256
NC, NS, L = 2, 16, 16
NW = NC * NS
EP = 800768
EPT = EP // NW
EPH = EP // NC
EPHT = EPH // NS
NPAD = 51200
NPT = NPAD // NW
ND = 50048
NDT = ND // NS
GD = 1152
GDT = GD // NS
B1 = 32
B2 = 64
B3 = 80
NCH = HP // L

_mesh = plsc.VectorSubcoreMesh(
    core_axis_name="c", subcore_axis_name="s", num_cores=NC, num_subcores=NS)
_sc_params = pltpu.CompilerParams(needs_layout_passes=False)


def _lrelu(v):
  return jnp.maximum(v, 0.01 * v)


def _alpha_gate_body(src_h, dst_h, t1_h, e1_h, r1_h, attl_h,
                     e_out, den_out,
                     r1v, attlv, sbuf, dbuf, tgbuf, egbuf, ebuf, trsc, obuf,
                     den_sh, sem):
  c = lax.axis_index("c")
  s = lax.axis_index("s")
  w = s * NC + c
  base = w * EPT

  def zfill(i, carry):
    obuf[pl.ds(i * L, L)] = jnp.zeros((L,), jnp.float32)
    return carry

  lax.fori_loop(0, NDT // L, zfill, 0)
  pltpu.sync_copy(obuf, den_sh.at[pl.ds(s * NDT, NDT)])
  pltpu.sync_copy(r1_h.at[pl.ds(0, ND)], r1v)
  pltpu.sync_copy(attl_h, attlv)
  plsc.subcore_barrier()

  iota = lax.iota(jnp.int32, L)

  def blk(bi, carry):
    off = base + bi * B1
    pltpu.sync_copy(src_h.at[pl.ds(off, B1)], sbuf)
    pltpu.sync_copy(dst_h.at[pl.ds(off, B1)], dbuf)
    pltpu.async_copy(t1_h.at[sbuf], tgbuf, sem).wait()
    pltpu.sync_copy(e1_h.at[pl.ds(off, B1)], egbuf)
    for g in range(B1 // L):
      for e in range(L):
        r = g * L + e
        acc = jnp.zeros((L,), jnp.float32)
        for v in range(NCH):
          t = tgbuf[r, pl.ds(v * L, L)] + egbuf[r, pl.ds(v * L, L)]
          acc = acc + _lrelu(t) * attlv[pl.ds(v * L, L)]
        trsc[pl.ds(e * L, L)] = acc
      tot = jnp.zeros((L,), jnp.float32)
      for k in range(L):
        tot = tot + plsc.load_gather(trsc, [iota * L + k])
      d16 = dbuf[pl.ds(g * L, L)]
      al = _lrelu(tot + plsc.load_gather(r1v, [d16]))
      ebuf[pl.ds(g * L, L)] = jnp.exp(al)
    pltpu.sync_copy(ebuf, e_out.at[pl.ds(off, B1)])
    pltpu.sync_copy(ebuf, den_sh.at[dbuf], add=True)
    return carry

  lax.fori_loop(0, EPT // B1, blk, 0)
  plsc.subcore_barrier()
  pltpu.sync_copy(den_sh.at[pl.ds(s * NDT, NDT)], obuf)
  pltpu.sync_copy(obuf, den_out.at[pl.ds(c * ND + s * NDT, NDT)])


def _sc_alpha_gate(srcp, dstp, t1, e1, r1, attl):
  return pl.kernel(
      _alpha_gate_body,
      out_type=(jax.ShapeDtypeStruct((EP,), jnp.float32),
                jax.ShapeDtypeStruct((NC * ND,), jnp.float32)),
      mesh=_mesh,
      compiler_params=_sc_params,
      scratch_types=[
          pltpu.VMEM((ND,), jnp.float32),
          pltpu.VMEM((HP,), jnp.float32),
          pltpu.VMEM((B1,), jnp.int32),
          pltpu.VMEM((B1,), jnp.int32),
          pltpu.VMEM((B1, HP), jnp.float32),
          pltpu.VMEM((B1, HP), jnp.float32),
          pltpu.VMEM((B1,), jnp.float32),
          pltpu.VMEM((L * L,), jnp.float32),
          pltpu.VMEM((NDT,), jnp.float32),
          pltpu.VMEM_SHARED((ND,), jnp.float32),
          pltpu.SemaphoreType.DMA,
      ],
  )(srcp, dstp, t1, e1, r1, attl)


def _alpha_gat_body(src_h, dst_h, as_h, ad_h,
                    e_out, den_out,
                    asv, adv, sbuf, dbuf, ebuf, obuf, den_sh):
  c = lax.axis_index("c")
  s = lax.axis_index("s")
  w = s * NC + c
  base = w * EPT

  def zfill(i, carry):
    obuf[pl.ds(i * L, L)] = jnp.zeros((L,), jnp.float32)
    return carry

  lax.fori_loop(0, NDT // L, zfill, 0)
  pltpu.sync_copy(obuf, den_sh.at[pl.ds(s * NDT, NDT)])
  pltpu.sync_copy(as_h.at[pl.ds(0, ND)], asv)
  pltpu.sync_copy(ad_h.at[pl.ds(0, ND)], adv)
  plsc.subcore_barrier()

  def blk(bi, carry):
    off = base + bi * B2
    pltpu.sync_copy(src_h.at[pl.ds(off, B2)], sbuf)
    pltpu.sync_copy(dst_h.at[pl.ds(off, B2)], dbuf)
    for g in range(B2 // L):
      s16 = sbuf[pl.ds(g * L, L)]
      d16 = dbuf[pl.ds(g * L, L)]
      al = _lrelu(plsc.load_gather(asv, [s16]) + plsc.load_gather(adv, [d16]))
      ebuf[pl.ds(g * L, L)] = jnp.exp(al)
    pltpu.sync_copy(ebuf, e_out.at[pl.ds(off, B2)])
    pltpu.sync_copy(ebuf, den_sh.at[dbuf], add=True)
    return carry

  lax.fori_loop(0, EPT // B2, blk, 0)
  plsc.subcore_barrier()
  pltpu.sync_copy(den_sh.at[pl.ds(s * NDT, NDT)], obuf)
  pltpu.sync_copy(obuf, den_out.at[pl.ds(c * ND + s * NDT, NDT)])


def _sc_alpha_gat(srcp, dstp, asr, adr):
  return pl.kernel(
      _alpha_gat_body,
      out_type=(jax.ShapeDtypeStruct((EP,), jnp.float32),
                jax.ShapeDtypeStruct((NC * ND,), jnp.float32)),
      mesh=_mesh,
      compiler_params=_sc_params,
      scratch_types=[
          pltpu.VMEM((ND,), jnp.float32),
          pltpu.VMEM((ND,), jnp.float32),
          pltpu.VMEM((B2,), jnp.int32),
          pltpu.VMEM((B2,), jnp.int32),
          pltpu.VMEM((B2,), jnp.float32),
          pltpu.VMEM((NDT,), jnp.float32),
          pltpu.VMEM_SHARED((ND,), jnp.float32),
      ],
  )(srcp, dstp, asr, adr)


QS = ND // 4
SEG = 2176
SEGV = SEG // L
NSEG = EP // SEG
SLIV = 784
DSEG = 6256
CSEG = 1088


def _coeff_body(dst_h, e_h, den_h, c_out, dv, tmpv, dseg, eseg, cseg):
  c = lax.axis_index("c")
  s = lax.axis_index("s")
  w = s * NC + c
  base = w * EPT

  pltpu.sync_copy(den_h.at[pl.ds(0, ND)], dv)

  def dadd(k, carry):
    pltpu.sync_copy(den_h.at[pl.ds(ND + k * DSEG, DSEG)], tmpv)

    def a2(i, carry2):
      dv[pl.ds(k * DSEG + i * L, L)] = (dv[pl.ds(k * DSEG + i * L, L)]
                                        + tmpv[pl.ds(i * L, L)])
      return carry2

    lax.fori_loop(0, DSEG // L, a2, 0)
    return carry

  lax.fori_loop(0, ND // DSEG, dadd, 0)

  def seg(k, carry):
    off = base + k * CSEG
    pltpu.sync_copy(dst_h.at[pl.ds(off, CSEG)], dseg)
    pltpu.sync_copy(e_h.at[pl.ds(off, CSEG)], eseg)

    def inner(i, carry2):
      d16 = dseg[pl.ds(i * L, L)]
      dn = plsc.load_gather(dv, [d16])
      cseg[pl.ds(i * L, L)] = eseg[pl.ds(i * L, L)] / (dn + 1e-16)
      return carry2

    lax.fori_loop(0, CSEG // L, inner, 0)
    pltpu.sync_copy(cseg, c_out.at[pl.ds(off, CSEG)])
    return carry

  lax.fori_loop(0, EPT // CSEG, seg, 0)


def _sc_coeff(dstp, e, den2):
  return pl.kernel(
      _coeff_body,
      out_type=jax.ShapeDtypeStruct((EP,), jnp.float32),
      mesh=_mesh,
      compiler_params=_sc_params,
      scratch_types=[
          pltpu.VMEM((ND,), jnp.float32),
          pltpu.VMEM((DSEG,), jnp.float32),
          pltpu.VMEM((CSEG,), jnp.int32),
          pltpu.VMEM((CSEG,), jnp.float32),
          pltpu.VMEM((CSEG,), jnp.float32),
      ],
  )(dstp, e, den2)


def _aggregate_body(src_h, dst_h, cf_h, tab2_h,
                    h_out,
                    sseg, dseg, cseg, slist, dllist, cflist,
                    idxb, dlb, cfb, rows, acc, sem):
  c = lax.axis_index("c")
  s = lax.axis_index("s")
  iota = lax.iota(jnp.int32, L)
  coff = c * NPAD
  zv = jnp.zeros((L,), jnp.float32)

  def do_block(b, rem):
    for gs in range(4):
      sl = pl.ds(b * 64 + gs * L, L)
      jv = gs * L + iota
      mv = jv < rem
      idxb[pl.ds(gs * L, L)] = jnp.where(mv, slist[sl], 0) + coff
      dlb[pl.ds(gs * L, L)] = jnp.where(mv, dllist[sl], 0)
      cfb[pl.ds(gs * L, L)] = jnp.where(mv, cflist[sl], 0.0)
    pltpu.async_copy(tab2_h.at[idxb], rows, sem).wait()

    def accg(g, carry):
      dl16 = dlb[pl.ds(g * L, L)]
      cf16 = cfb[pl.ds(g * L, L)]
      for e in range(L):
        dl = dl16[e]
        cf = cf16[e]
        for v in range(8):
          acc[dl, pl.ds(v * L, L)] = (acc[dl, pl.ds(v * L, L)]
                                      + rows[g * L + e, pl.ds(v * L, L)] * cf)
      return carry

    lax.fori_loop(0, 4, accg, 0)

  for p in range(4):
    g = p * NS + s
    base = g * SLIV
    hi = jnp.where(g == 63, N + 48, base + SLIV) - base

    def zacc(i, carry):
      for v in range(8):
        acc[i, pl.ds(v * L, L)] = zv
      return carry

    lax.fori_loop(0, SLIV, zacc, 0)

    def seg_loop(sg, cnt):
      segoff = sg * SEG
      pltpu.sync_copy(src_h.at[pl.ds(segoff, SEG)], sseg)
      pltpu.sync_copy(dst_h.at[pl.ds(segoff, SEG)], dseg)
      pltpu.sync_copy(cf_h.at[pl.ds(segoff, SEG)], cseg)

      def comp(i, cnt2):
        d16 = dseg[pl.ds(i * L, L)]
        dl = d16 - base
        m = (dl >= 0) & (dl < hi)
        pos = plsc.cumsum(jnp.where(m, 1, 0))
        tgt = jnp.where(m, cnt2 + pos - 1, SEG + 127)
        plsc.store_scatter(slist, [tgt], sseg[pl.ds(i * L, L)], mask=m)
        plsc.store_scatter(dllist, [tgt], dl, mask=m)
        plsc.store_scatter(cflist, [tgt], cseg[pl.ds(i * L, L)], mask=m)
        return cnt2 + pos[L - 1]

      cnt = lax.fori_loop(0, SEGV, comp, cnt)
      nfull = cnt // 64

      def flush(b, carry):
        do_block(b, 64)
        return carry

      lax.fori_loop(0, nfull, flush, 0)
      rem = cnt - nfull * 64

      def mvrem(j, carry):
        slist[pl.ds(j * L, L)] = slist[pl.ds(nfull * 64 + j * L, L)]
        dllist[pl.ds(j * L, L)] = dllist[pl.ds(nfull * 64 + j * L, L)]
        cflist[pl.ds(j * L, L)] = cflist[pl.ds(nfull * 64 + j * L, L)]
        return carry

      lax.fori_loop(0, 4, mvrem, 0)
      return rem

    remf = lax.fori_loop(0, NSEG, seg_loop, 0)

    @pl.when(remf > 0)
    def _():
      do_block(0, remf)

    @pl.when(g == 63)
    def _():
      pltpu.sync_copy(acc.at[pl.ds(0, ND - 63 * SLIV)],
                      h_out.at[pl.ds(c * ND + 63 * SLIV, ND - 63 * SLIV)])

    @pl.when(g < 63)
    def _():
      pltpu.sync_copy(acc.at[pl.ds(0, SLIV)],
                      h_out.at[pl.ds(c * ND + g * SLIV, SLIV)])


def _sc_aggregate(srcp, dstp, cf, table):
  tab2 = jnp.concatenate([table[:, :128], table[:, 128:]], axis=0)
  return pl.kernel(
      _aggregate_body,
      out_type=jax.ShapeDtypeStruct((NC * ND, 128), jnp.float32),
      mesh=_mesh,
      compiler_params=_sc_params,
      scratch_types=[
          pltpu.VMEM((SEG,), jnp.int32),
          pltpu.VMEM((SEG,), jnp.int32),
          pltpu.VMEM((SEG,), jnp.float32),
          pltpu.VMEM((SEG + 128,), jnp.int32),
          pltpu.VMEM((SEG + 128,), jnp.int32),
          pltpu.VMEM((SEG + 128,), jnp.float32),
          pltpu.VMEM((64,), jnp.int32),
          pltpu.VMEM((64,), jnp.int32),
          pltpu.VMEM((64,), jnp.float32),
          pltpu.VMEM((64, 128), jnp.float32),
          pltpu.VMEM((SLIV, 128), jnp.float32),
          pltpu.SemaphoreType.DMA,
      ],
  )(srcp, dstp, cf, tab2)


def _readout_alpha_body(b_h, ss_h, td_h,
                        eg_out, dg_out,
                        bv, ssv, tdv, egv, bchunk, echunk, obuf, dg_sh):
  c = lax.axis_index("c")
  s = lax.axis_index("s")
  w = s * NC + c
  base = w * NPT

  def zfill(i, carry):
    obuf[pl.ds(i * L, L)] = jnp.zeros((L,), jnp.float32)
    return carry

  lax.fori_loop(0, GDT // L, zfill, 0)
  pltpu.sync_copy(obuf, dg_sh.at[pl.ds(s * GDT, GDT)])
  pltpu.sync_copy(b_h.at[pl.ds(base, NPT)], bv)
  pltpu.sync_copy(ss_h.at[pl.ds(base, NPT)], ssv)
  pltpu.sync_copy(td_h, tdv.at[pl.ds(0, G)])
  plsc.subcore_barrier()

  for i in range(NPT // L):
    b16 = bv[pl.ds(i * L, L)]
    al = _lrelu(ssv[pl.ds(i * L, L)] + plsc.load_gather(tdv, [b16]))
    egv[pl.ds(i * L, L)] = jnp.exp(al)

  pltpu.sync_copy(egv, eg_out.at[pl.ds(base, NPT)])

  def scat(k, carry):
    for j in range(5):
      bchunk[pl.ds(j * L, L)] = bv[pl.ds(k * 80 + j * L, L)]
      echunk[pl.ds(j * L, L)] = egv[pl.ds(k * 80 + j * L, L)]
    pltpu.sync_copy(echunk, dg_sh.at[bchunk], add=True)
    return carry

  lax.fori_loop(0, NPT // 80, scat, 0)
  plsc.subcore_barrier()
  pltpu.sync_copy(dg_sh.at[pl.ds(s * GDT, GDT)], obuf)
  pltpu.sync_copy(obuf, dg_out.at[pl.ds(c * GD + s * GDT, GDT)])


def _sc_readout_alpha(batchp, ss, td):
  return pl.kernel(
      _readout_alpha_body,
      out_type=(jax.ShapeDtypeStruct((NPAD,), jnp.float32),
                jax.ShapeDtypeStruct((NC * GD,), jnp.float32)),
      mesh=_mesh,
      compiler_params=_sc_params,
      scratch_types=[
          pltpu.VMEM((NPT,), jnp.int32),
          pltpu.VMEM((NPT,), jnp.float32),
          pltpu.VMEM((GD,), jnp.float32),
          pltpu.VMEM((NPT,), jnp.float32),
          pltpu.VMEM((80,), jnp.int32),
          pltpu.VMEM((80,), jnp.float32),
          pltpu.VMEM((GDT,), jnp.float32),
          pltpu.VMEM_SHARED((GD,), jnp.float32),
      ],
  )(batchp, ss, td)


def _readout_agg_body(b_h, eg_h, dg_h, tab_h,
                      g_out,
                      dg0v, dg1v, bb, egb, cb, rowsa, rowsb, zbuf,
                      gacca, gaccb, sem):
  c = lax.axis_index("c")
  s = lax.axis_index("s")
  w = s * NC + c
  base = w * NPT

  pltpu.sync_copy(dg_h.at[pl.ds(0, GD)], dg0v)
  pltpu.sync_copy(dg_h.at[pl.ds(GD, GD)], dg1v)
  zv = jnp.zeros((L,), jnp.float32)

  def zinit(i, carry):
    for v in range(8):
      zbuf[i, pl.ds(v * L, L)] = zv
    return carry

  lax.fori_loop(0, GDT, zinit, 0)
  pltpu.sync_copy(zbuf, gacca.at[pl.ds(s * GDT, GDT)])
  pltpu.sync_copy(zbuf, gaccb.at[pl.ds(s * GDT, GDT)])
  plsc.subcore_barrier()

  def blk(bi, carry):
    off = base + bi * B3
    pltpu.sync_copy(b_h.at[pl.ds(off, B3)], bb)
    pltpu.sync_copy(eg_h.at[pl.ds(off, B3)], egb)
    pltpu.sync_copy(tab_h.at[pl.ds(off, B3), pl.ds(0, 128)], rowsa)
    pltpu.sync_copy(tab_h.at[pl.ds(off, B3), pl.ds(128, 128)], rowsb)
    for i in range(B3 // L):
      b16 = bb[pl.ds(i * L, L)]
      den = (plsc.load_gather(dg0v, [b16]) + plsc.load_gather(dg1v, [b16])
             + 1e-16)
      cb[pl.ds(i * L, L)] = egb[pl.ds(i * L, L)] / den

    def scale(g, carry2):
      c16 = cb[pl.ds(g * L, L)]
      for e in range(L):
        r = g * L + e
        for v in range(8):
          rowsa[r, pl.ds(v * L, L)] = rowsa[r, pl.ds(v * L, L)] * c16[e]
          rowsb[r, pl.ds(v * L, L)] = rowsb[r, pl.ds(v * L, L)] * c16[e]
      return carry2

    lax.fori_loop(0, B3 // L, scale, 0)
    pltpu.sync_copy(rowsa, gacca.at[bb], add=True)
    pltpu.sync_copy(rowsb, gaccb.at[bb], add=True)
    return carry

  lax.fori_loop(0, NPT // B3, blk, 0)

  plsc.subcore_barrier()
  pltpu.sync_copy(gacca.at[pl.ds(s * GDT, GDT)], zbuf)
  pltpu.sync_copy(zbuf, g_out.at[pl.ds(c * GD + s * GDT, GDT), pl.ds(0, 128)])
  pltpu.sync_copy(gaccb.at[pl.ds(s * GDT, GDT)], zbuf)
  pltpu.sync_copy(zbuf,
                  g_out.at[pl.ds(c * GD + s * GDT, GDT), pl.ds(128, 128)])


def _sc_readout_agg(batchp, eg, dg2, table):
  return pl.kernel(
      _readout_agg_body,
      out_type=jax.ShapeDtypeStruct((NC * GD, HP), jnp.float32),
      mesh=_mesh,
      compiler_params=_sc_params,
      scratch_types=[
          pltpu.VMEM((GD,), jnp.float32),
          pltpu.VMEM((GD,), jnp.float32),
          pltpu.VMEM((B3,), jnp.int32),
          pltpu.VMEM((B3,), jnp.float32),
          pltpu.VMEM((B3,), jnp.float32),
          pltpu.VMEM((B3, 128), jnp.float32),
          pltpu.VMEM((B3, 128), jnp.float32),
          pltpu.VMEM((GDT, 128), jnp.float32),
          pltpu.VMEM_SHARED((GD, 128), jnp.float32),
          pltpu.VMEM_SHARED((GD, 128), jnp.float32),
          pltpu.SemaphoreType.DMA,
      ],
  )(batchp, eg, dg2, table)


BM = 400
NBLK = NPAD // BM
NBLK_REAL = N // BM


def _dot(a, b):
  return jax.lax.dot_general(a, b, (((1,), (0,)), ((), ())),
                             preferred_element_type=jnp.float32)


def _tc_main1_body(x_r, w1t_r, b1_r, wxt_r, w2t_r, attr_r,
                   x1_o, t1_o, r1_o, w1v_o):
  x1 = _lrelu(_dot(x_r[...], w1t_r[...]) + b1_r[...])
  x1_o[...] = x1
  t1_o[...] = _dot(x1, wxt_r[...])
  r1_o[...] = _dot(x1, attr_r[...])
  w1v_o[...] = _dot(x1, w2t_r[...])


def _tc_main1(x, w1t, b1, wxt, w2t, attr):
  return pl.pallas_call(
      _tc_main1_body,
      grid=(NBLK,),
      in_specs=[
          pl.BlockSpec((BM, 39), lambda i: (jnp.minimum(i, NBLK_REAL - 1), 0)),
          pl.BlockSpec((39, HP), lambda i: (0, 0)),
          pl.BlockSpec((1, HP), lambda i: (0, 0)),
          pl.BlockSpec((HP, HP), lambda i: (0, 0)),
          pl.BlockSpec((HP, HP), lambda i: (0, 0)),
          pl.BlockSpec((HP, 1), lambda i: (0, 0)),
      ],
      out_specs=[
          pl.BlockSpec((BM, HP), lambda i: (i, 0)),
          pl.BlockSpec((BM, HP), lambda i: (i, 0)),
          pl.BlockSpec((BM, 1), lambda i: (i, 0)),
          pl.BlockSpec((BM, HP), lambda i: (i, 0)),
      ],
      out_shape=[
          jax.ShapeDtypeStruct((NPAD, HP), jnp.float32),
          jax.ShapeDtypeStruct((NPAD, HP), jnp.float32),
          jax.ShapeDtypeStruct((NPAD, 1), jnp.float32),
          jax.ShapeDtypeStruct((NPAD, HP), jnp.float32),
      ],
  )(x, w1t, b1, wxt, w2t, attr)


BME = 1024


def _tc_e1_body(ea_r, wet_r, e1_o):
  e1_o[...] = _dot(ea_r[...], wet_r[...])


def _tc_e1(eap, wet):
  return pl.pallas_call(
      _tc_e1_body,
      grid=(EP // BME,),
      in_specs=[
          pl.BlockSpec((BME, 16), lambda i: (i, 0)),
          pl.BlockSpec((16, HP), lambda i: (0, 0)),
      ],
      out_specs=pl.BlockSpec((BME, HP), lambda i: (i, 0)),
      out_shape=jax.ShapeDtypeStruct((EP, HP), jnp.float32),
  )(eap, wet)


def _gru_block(h, xprev, wiht_r, whht_r, bih_r, bhh_r):
  gi0 = _dot(h, wiht_r[0]) + bih_r[0, 0]
  gi1 = _dot(h, wiht_r[1]) + bih_r[0, 1]
  gi2 = _dot(h, wiht_r[2]) + bih_r[0, 2]
  gh0 = _dot(xprev, whht_r[0]) + bhh_r[0, 0]
  gh1 = _dot(xprev, whht_r[1]) + bhh_r[0, 1]
  gh2 = _dot(xprev, whht_r[2]) + bhh_r[0, 2]
  r = jax.nn.sigmoid(gi0 + gh0)
  z = jax.nn.sigmoid(gi1 + gh1)
  n = jnp.tanh(gi2 + r * gh2)
  return (1.0 - z) * n + z * xprev


def _elu(v):
  return jnp.where(v > 0, v, jnp.exp(jnp.minimum(v, 0.0)) - 1.0)


def _tc_gru0_body(h0_r, h1_r, x1_r, gb_r, wiht_r, whht_r, bih_r, bhh_r,
                  gatwt_r, gas_r, gad_r,
                  x2_o, as_o, ad_o, xt_o):
  hs = jnp.concatenate([h0_r[...], h1_r[...]], axis=1)
  h = _elu(hs + gb_r[...])
  x2 = jax.nn.relu(_gru_block(h, x1_r[...], wiht_r, whht_r, bih_r, bhh_r))
  x2_o[...] = x2
  xt = _dot(x2, gatwt_r[...])
  as_o[...] = _dot(xt, gas_r[...])
  ad_o[...] = _dot(xt, gad_r[...])
  xt_o[...] = xt


def _tc_gru0(h0, h1, x1p, gb, wiht, whht, bih, bhh, gatwt, gas, gad):
  return pl.pallas_call(
      _tc_gru0_body,
      grid=(NBLK,),
      in_specs=[
          pl.BlockSpec((BM, 128), lambda i: (jnp.minimum(i, NBLK_REAL - 1), 0)),
          pl.BlockSpec((BM, 128), lambda i: (jnp.minimum(i, NBLK_REAL - 1), 0)),
          pl.BlockSpec((BM, HP), lambda i: (jnp.minimum(i, NBLK_REAL - 1), 0)),
          pl.BlockSpec((1, HP), lambda i: (0, 0)),
          pl.BlockSpec((3, HP, HP), lambda i: (0, 0, 0)),
          pl.BlockSpec((3, HP, HP), lambda i: (0, 0, 0)),
          pl.BlockSpec((1, 3, HP), lambda i: (0, 0, 0)),
          pl.BlockSpec((1, 3, HP), lambda i: (0, 0, 0)),
          pl.BlockSpec((HP, HP), lambda i: (0, 0)),
          pl.BlockSpec((HP, 1), lambda i: (0, 0)),
          pl.BlockSpec((HP, 1), lambda i: (0, 0)),
      ],
      out_specs=[
          pl.BlockSpec((BM, HP), lambda i: (i, 0)),
          pl.BlockSpec((BM, 1), lambda i: (i, 0)),
          pl.BlockSpec((BM, 1), lambda i: (i, 0)),
          pl.BlockSpec((BM, HP), lambda i: (i, 0)),
      ],
      out_shape=[
          jax.ShapeDtypeStruct((NPAD, HP), jnp.float32),
          jax.ShapeDtypeStruct((NPAD, 1), jnp.float32),
          jax.ShapeDtypeStruct((NPAD, 1), jnp.float32),
          jax.ShapeDtypeStruct((NPAD, HP), jnp.float32),
      ],
  )(h0, h1, x1p, gb, wiht, whht, bih, bhh, gatwt, gas, gad)


def _tc_gru1_body(h0_r, h1_r, x2_r, gb_r, wiht_r, whht_r, bih_r, bhh_r,
                  molwt_r, mas_r,
                  x3_o, xs_o, ss_o):
  hs = jnp.concatenate([h0_r[...], h1_r[...]], axis=1)
  h = _elu(hs + gb_r[...])
  x3 = jax.nn.relu(_gru_block(h, x2_r[...], wiht_r, whht_r, bih_r, bhh_r))
  x3_o[...] = x3
  xs = _dot(x3, molwt_r[...])
  xs_o[...] = xs
  ss_o[...] = _dot(xs, mas_r[...])


def _tc_gru1(h0, h1, x2p, gb, wiht, whht, bih, bhh, molwt, mas):
  return pl.pallas_call(
      _tc_gru1_body,
      grid=(NBLK,),
      in_specs=[
          pl.BlockSpec((BM, 128), lambda i: (jnp.minimum(i, NBLK_REAL - 1), 0)),
          pl.BlockSpec((BM, 128), lambda i: (jnp.minimum(i, NBLK_REAL - 1), 0)),
          pl.BlockSpec((BM, HP), lambda i: (jnp.minimum(i, NBLK_REAL - 1), 0)),
          pl.BlockSpec((1, HP), lambda i: (0, 0)),
          pl.BlockSpec((3, HP, HP), lambda i: (0, 0, 0)),
          pl.BlockSpec((3, HP, HP), lambda i: (0, 0, 0)),
          pl.BlockSpec((1, 3, HP), lambda i: (0, 0, 0)),
          pl.BlockSpec((1, 3, HP), lambda i: (0, 0, 0)),
          pl.BlockSpec((HP, HP), lambda i: (0, 0)),
          pl.BlockSpec((HP, 1), lambda i: (0, 0)),
      ],
      out_specs=[
          pl.BlockSpec((BM, HP), lambda i: (i, 0)),
          pl.BlockSpec((BM, HP), lambda i: (i, 0)),
          pl.BlockSpec((BM, 1), lambda i: (i, 0)),
      ],
      out_shape=[
          jax.ShapeDtypeStruct((NPAD, HP), jnp.float32),
          jax.ShapeDtypeStruct((NPAD, HP), jnp.float32),
          jax.ShapeDtypeStruct((NPAD, 1), jnp.float32),
      ],
  )(h0, h1, x2p, gb, wiht, whht, bih, bhh, molwt, mas)


def _tc_pool_body(p_r, molwt_r, mad_r, out_o, td_o):
  out0 = jax.nn.relu(p_r[0, :G, :] + p_r[1, :G, :])
  out_o[...] = out0
  td_o[...] = _dot(_dot(out0, molwt_r[...]), mad_r[...])


def _tc_pool(pool2, molwt, mad):
  return pl.pallas_call(
      _tc_pool_body,
      grid=(1,),
      in_specs=[
          pl.BlockSpec((NC, GD, HP), lambda i: (0, 0, 0)),
          pl.BlockSpec((HP, HP), lambda i: (0, 0)),
          pl.BlockSpec((HP, 1), lambda i: (0, 0)),
      ],
      out_specs=[
          pl.BlockSpec((G, HP), lambda i: (0, 0)),
          pl.BlockSpec((G, 1), lambda i: (0, 0)),
      ],
      out_shape=[
          jax.ShapeDtypeStruct((G, HP), jnp.float32),
          jax.ShapeDtypeStruct((G, 1), jnp.float32),
      ],
  )(pool2, molwt, mad)


def _tc_mgru_body(g_r, op_r, mb_r, wiht_r, whht_r, bih_r, bhh_r,
                  molwt_r, mad_r, l2t_r, l2b_r,
                  out_o, td_o, y_o):
  h = _elu(g_r[0, :G, :] + g_r[1, :G, :] + mb_r[...])
  outn = jax.nn.relu(_gru_block(h, op_r[...], wiht_r, whht_r, bih_r, bhh_r))
  out_o[...] = outn
  td_o[...] = _dot(_dot(outn, molwt_r[...]), mad_r[...])
  y_o[...] = _dot(outn, l2t_r[...]) + l2b_r[...]


def _tc_mgru(gs, outp, mb, wiht, whht, bih, bhh, molwt, mad, l2t, l2b):
  return pl.pallas_call(
      _tc_mgru_body,
      grid=(1,),
      in_specs=[
          pl.BlockSpec((NC, GD, HP), lambda i: (0, 0, 0)),
          pl.BlockSpec((G, HP), lambda i: (0, 0)),
          pl.BlockSpec((1, HP), lambda i: (0, 0)),
          pl.BlockSpec((3, HP, HP), lambda i: (0, 0, 0)),
          pl.BlockSpec((3, HP, HP), lambda i: (0, 0, 0)),
          pl.BlockSpec((1, 3, HP), lambda i: (0, 0, 0)),
          pl.BlockSpec((1, 3, HP), lambda i: (0, 0, 0)),
          pl.BlockSpec((HP, HP), lambda i: (0, 0)),
          pl.BlockSpec((HP, 1), lambda i: (0, 0)),
          pl.BlockSpec((HP, 1), lambda i: (0, 0)),
          pl.BlockSpec((1, 1), lambda i: (0, 0)),
      ],
      out_specs=[
          pl.BlockSpec((G, HP), lambda i: (0, 0)),
          pl.BlockSpec((G, 1), lambda i: (0, 0)),
          pl.BlockSpec((G, 1), lambda i: (0, 0)),
      ],
      out_shape=[
          jax.ShapeDtypeStruct((G, HP), jnp.float32),
          jax.ShapeDtypeStruct((G, 1), jnp.float32),
          jax.ShapeDtypeStruct((G, 1), jnp.float32),
      ],
  )(gs, outp, mb, wiht, whht, bih, bhh, molwt, mad, l2t, l2b)


def _padw(w, rows, cols):
  return jnp.pad(w, ((0, rows - w.shape[0]), (0, cols - w.shape[1])))


def _pack_gru(p, pre):
  wih = p[pre + "_Wih"]
  whh = p[pre + "_Whh"]
  bih = p[pre + "_bih"]
  bhh = p[pre + "_bhh"]
  wiht = jnp.stack([_padw(wih[k * H:(k + 1) * H].T, HP, HP) for k in range(3)])
  whht = jnp.stack([_padw(whh[k * H:(k + 1) * H].T, HP, HP) for k in range(3)])
  biht = jnp.stack([jnp.pad(bih[k * H:(k + 1) * H], (0, HP - H))
                    for k in range(3)])[None]
  bhht = jnp.stack([jnp.pad(bhh[k * H:(k + 1) * H], (0, HP - H))
                    for k in range(3)])[None]
  return wiht, whht, biht, bhht


@jax.jit
def _run(x, edge_index, edge_attr, batch, p):
  src, dst = edge_index[0], edge_index[1]
  padn = EP - E
  srcp = jnp.concatenate([src, (jnp.arange(padn, dtype=jnp.int32) * 61) % N])
  dstp = jnp.concatenate([dst, jnp.full((padn,), N, jnp.int32)])
  batchp = jnp.concatenate(
      [batch, jnp.full((NPAD - N,), G, jnp.int32)]).astype(jnp.int32)
  eap = jnp.pad(edge_attr, ((0, EP - E), (0, 16 - 10)))
  ones_np = jnp.ones((NPAD,), jnp.float32)
  den_one = jnp.concatenate([jnp.ones((GD,), jnp.float32),
                             jnp.zeros((GD,), jnp.float32)])

  w1t = jnp.pad(p["lin1_W"].T, ((0, 0), (0, HP - H)))
  b1 = jnp.pad(p["lin1_b"], (0, HP - H))[None]
  wxt = _padw(p["gate_lin1_W"][:, :H].T, HP, HP)
  wet = jnp.pad(p["gate_lin1_W"][:, H:].T, ((0, 16 - 10), (0, HP - H)))
  w2t = _padw(p["gate_lin2_W"].T, HP, HP)
  attl = jnp.pad(p["gate_att_l"], (0, HP - H))
  attr = jnp.pad(p["gate_att_r"], (0, HP - H))[:, None]
  gb = jnp.pad(p["gate_bias"], (0, HP - H))[None]
  gatwt = _padw(p["gat_W"].T, HP, HP)
  gas = jnp.pad(p["gat_att_src"], (0, HP - H))[:, None]
  gad = jnp.pad(p["gat_att_dst"], (0, HP - H))[:, None]
  gatb = jnp.pad(p["gat_bias"], (0, HP - H))[None]
  molwt = _padw(p["mol_W"].T, HP, HP)
  mas = jnp.pad(p["mol_att_src"], (0, HP - H))[:, None]
  mad = jnp.pad(p["mol_att_dst"], (0, HP - H))[:, None]
  molb = jnp.pad(p["mol_bias"], (0, HP - H))[None]
  l2t = jnp.pad(p["lin2_W"].T, ((0, HP - H), (0, 0)))
  l2b = p["lin2_b"][None]
  g0 = _pack_gru(p, "gru0")
  g1 = _pack_gru(p, "gru1")
  mg = _pack_gru(p, "mgru")

  x1p, t1, r1, w1v = _tc_main1(x, w1t, b1, wxt, w2t, attr)
  e1 = _tc_e1(eap, wet)
  ev, den2 = _sc_alpha_gate(srcp, dstp, t1, e1, r1.reshape(NPAD), attl)
  cf1 = _sc_coeff(dstp, ev, den2)
  h1f = _sc_aggregate(srcp, dstp, cf1, w1v)
  x2p, asr, adr, xtp = _tc_gru0(h1f[:ND], h1f[ND:], x1p, gb, *g0,
                                gatwt, gas, gad)
  ev2, den2b = _sc_alpha_gat(srcp, dstp, asr.reshape(NPAD),
                             adr.reshape(NPAD))
  cf2 = _sc_coeff(dstp, ev2, den2b)
  h2f = _sc_aggregate(srcp, dstp, cf2, xtp)
  x3p, xsp, ssr = _tc_gru1(h2f[:ND], h2f[ND:], x2p, gatb, *g1, molwt, mas)
  pool2 = _sc_readout_agg(batchp, ones_np, den_one, x3p).reshape(NC, GD, HP)
  outp, td = _tc_pool(pool2, molwt, mad)
  y = None
  for _ in range(2):
    eg, dg2 = _sc_readout_alpha(batchp, ssr.reshape(NPAD), td.reshape(G))
    gs = _sc_readout_agg(batchp, eg, dg2, xsp).reshape(NC, GD, HP)
    outp, td, y = _tc_mgru(gs, outp, molb, *mg, molwt, mad, l2t, l2b)
  return y


def kernel(x, edge_index, edge_attr, batch, params):
  return _run(x, edge_index, edge_attr, batch, params)


# speedup vs baseline: 1.9573x; 1.0005x over previous
import jax
import jax.numpy as jnp
from jax import lax
from jax.experimental import pallas as pl
from jax.experimental.pallas import tpu as pltpu
from jax.experimental.pallas import tpu_sc as plsc

N = 50000
E = 800000
G = 1024
H = 200
HP = 256
NC, NS, L = 2, 16, 16
NW = NC * NS
EP = 800768
EPT = EP // NW
EPH = EP // NC
EPHT = EPH // NS
NPAD = 51200
NPT = NPAD // NW
ND = 50048
NDT = ND // NS
GD = 1152
GDT = GD // NS
B1 = 32
B2 = 64
B3 = 80
NCH = HP // L

_mesh = plsc.VectorSubcoreMesh(
    core_axis_name="c", subcore_axis_name="s", num_cores=NC, num_subcores=NS)
_sc_params = pltpu.CompilerParams(needs_layout_passes=False)


def _lrelu(v):
  return jnp.maximum(v, 0.01 * v)


def _alpha_gate_body(src_h, dst_h, t1_h, e1_h, r1_h, attl_h,
                     e_out, den_out,
                     r1v, attlv, sbuf, dbuf, tgbuf, egbuf, ebuf, trsc, obuf,
                     den_sh, sem):
  c = lax.axis_index("c")
  s = lax.axis_index("s")
  w = s * NC + c
  base = w * EPT

  def zfill(i, carry):
    obuf[pl.ds(i * L, L)] = jnp.zeros((L,), jnp.float32)
    return carry

  lax.fori_loop(0, NDT // L, zfill, 0)
  pltpu.sync_copy(obuf, den_sh.at[pl.ds(s * NDT, NDT)])
  pltpu.sync_copy(r1_h.at[pl.ds(0, ND)], r1v)
  pltpu.sync_copy(attl_h, attlv)
  plsc.subcore_barrier()

  iota = lax.iota(jnp.int32, L)

  def blk(bi, carry):
    off = base + bi * B1
    pltpu.sync_copy(src_h.at[pl.ds(off, B1)], sbuf)
    pltpu.sync_copy(dst_h.at[pl.ds(off, B1)], dbuf)
    pltpu.async_copy(t1_h.at[sbuf], tgbuf, sem).wait()
    pltpu.sync_copy(e1_h.at[pl.ds(off, B1)], egbuf)
    for g in range(B1 // L):
      for e in range(L):
        r = g * L + e
        acc = jnp.zeros((L,), jnp.float32)
        for v in range(NCH):
          t = tgbuf[r, pl.ds(v * L, L)] + egbuf[r, pl.ds(v * L, L)]
          acc = acc + _lrelu(t) * attlv[pl.ds(v * L, L)]
        trsc[pl.ds(e * L, L)] = acc
      tot = jnp.zeros((L,), jnp.float32)
      for k in range(L):
        tot = tot + plsc.load_gather(trsc, [iota * L + k])
      d16 = dbuf[pl.ds(g * L, L)]
      al = _lrelu(tot + plsc.load_gather(r1v, [d16]))
      ebuf[pl.ds(g * L, L)] = jnp.exp(al)
    pltpu.sync_copy(ebuf, e_out.at[pl.ds(off, B1)])
    pltpu.sync_copy(ebuf, den_sh.at[dbuf], add=True)
    return carry

  lax.fori_loop(0, EPT // B1, blk, 0)
  plsc.subcore_barrier()
  pltpu.sync_copy(den_sh.at[pl.ds(s * NDT, NDT)], obuf)
  pltpu.sync_copy(obuf, den_out.at[pl.ds(c * ND + s * NDT, NDT)])


def _sc_alpha_gate(srcp, dstp, t1, e1, r1, attl):
  return pl.kernel(
      _alpha_gate_body,
      out_type=(jax.ShapeDtypeStruct((EP,), jnp.float32),
                jax.ShapeDtypeStruct((NC * ND,), jnp.float32)),
      mesh=_mesh,
      compiler_params=_sc_params,
      scratch_types=[
          pltpu.VMEM((ND,), jnp.float32),
          pltpu.VMEM((HP,), jnp.float32),
          pltpu.VMEM((B1,), jnp.int32),
          pltpu.VMEM((B1,), jnp.int32),
          pltpu.VMEM((B1, HP), jnp.float32),
          pltpu.VMEM((B1, HP), jnp.float32),
          pltpu.VMEM((B1,), jnp.float32),
          pltpu.VMEM((L * L,), jnp.float32),
          pltpu.VMEM((NDT,), jnp.float32),
          pltpu.VMEM_SHARED((ND,), jnp.float32),
          pltpu.SemaphoreType.DMA,
      ],
  )(srcp, dstp, t1, e1, r1, attl)


def _alpha_gat_body(src_h, dst_h, as_h, ad_h,
                    e_out, den_out,
                    asv, adv, sbuf, dbuf, ebuf, obuf, den_sh):
  c = lax.axis_index("c")
  s = lax.axis_index("s")
  w = s * NC + c
  base = w * EPT

  def zfill(i, carry):
    obuf[pl.ds(i * L, L)] = jnp.zeros((L,), jnp.float32)
    return carry

  lax.fori_loop(0, NDT // L, zfill, 0)
  pltpu.sync_copy(obuf, den_sh.at[pl.ds(s * NDT, NDT)])
  pltpu.sync_copy(as_h.at[pl.ds(0, ND)], asv)
  pltpu.sync_copy(ad_h.at[pl.ds(0, ND)], adv)
  plsc.subcore_barrier()

  def blk(bi, carry):
    off = base + bi * B2
    pltpu.sync_copy(src_h.at[pl.ds(off, B2)], sbuf)
    pltpu.sync_copy(dst_h.at[pl.ds(off, B2)], dbuf)
    for g in range(B2 // L):
      s16 = sbuf[pl.ds(g * L, L)]
      d16 = dbuf[pl.ds(g * L, L)]
      al = _lrelu(plsc.load_gather(asv, [s16]) + plsc.load_gather(adv, [d16]))
      ebuf[pl.ds(g * L, L)] = jnp.exp(al)
    pltpu.sync_copy(ebuf, e_out.at[pl.ds(off, B2)])
    pltpu.sync_copy(ebuf, den_sh.at[dbuf], add=True)
    return carry

  lax.fori_loop(0, EPT // B2, blk, 0)
  plsc.subcore_barrier()
  pltpu.sync_copy(den_sh.at[pl.ds(s * NDT, NDT)], obuf)
  pltpu.sync_copy(obuf, den_out.at[pl.ds(c * ND + s * NDT, NDT)])


def _sc_alpha_gat(srcp, dstp, asr, adr):
  return pl.kernel(
      _alpha_gat_body,
      out_type=(jax.ShapeDtypeStruct((EP,), jnp.float32),
                jax.ShapeDtypeStruct((NC * ND,), jnp.float32)),
      mesh=_mesh,
      compiler_params=_sc_params,
      scratch_types=[
          pltpu.VMEM((ND,), jnp.float32),
          pltpu.VMEM((ND,), jnp.float32),
          pltpu.VMEM((B2,), jnp.int32),
          pltpu.VMEM((B2,), jnp.int32),
          pltpu.VMEM((B2,), jnp.float32),
          pltpu.VMEM((NDT,), jnp.float32),
          pltpu.VMEM_SHARED((ND,), jnp.float32),
      ],
  )(srcp, dstp, asr, adr)


QS = ND // 4
SEG = 2176
SEGV = SEG // L
NSEG = EP // SEG
SLIV = 784
DSEG = 6256
CSEG = 1088


def _coeff_body(dst_h, e_h, den_h, c_out, dv, tmpv, dseg, eseg, cseg):
  c = lax.axis_index("c")
  s = lax.axis_index("s")
  w = s * NC + c
  base = w * EPT

  pltpu.sync_copy(den_h.at[pl.ds(0, ND)], dv)

  def dadd(k, carry):
    pltpu.sync_copy(den_h.at[pl.ds(ND + k * DSEG, DSEG)], tmpv)

    def a2(i, carry2):
      dv[pl.ds(k * DSEG + i * L, L)] = (dv[pl.ds(k * DSEG + i * L, L)]
                                        + tmpv[pl.ds(i * L, L)])
      return carry2

    lax.fori_loop(0, DSEG // L, a2, 0)
    return carry

  lax.fori_loop(0, ND // DSEG, dadd, 0)

  def seg(k, carry):
    off = base + k * CSEG
    pltpu.sync_copy(dst_h.at[pl.ds(off, CSEG)], dseg)
    pltpu.sync_copy(e_h.at[pl.ds(off, CSEG)], eseg)

    def inner(i, carry2):
      d16 = dseg[pl.ds(i * L, L)]
      dn = plsc.load_gather(dv, [d16])
      cseg[pl.ds(i * L, L)] = eseg[pl.ds(i * L, L)] / (dn + 1e-16)
      return carry2

    lax.fori_loop(0, CSEG // L, inner, 0)
    pltpu.sync_copy(cseg, c_out.at[pl.ds(off, CSEG)])
    return carry

  lax.fori_loop(0, EPT // CSEG, seg, 0)


def _sc_coeff(dstp, e, den2):
  return pl.kernel(
      _coeff_body,
      out_type=jax.ShapeDtypeStruct((EP,), jnp.float32),
      mesh=_mesh,
      compiler_params=_sc_params,
      scratch_types=[
          pltpu.VMEM((ND,), jnp.float32),
          pltpu.VMEM((DSEG,), jnp.float32),
          pltpu.VMEM((CSEG,), jnp.int32),
          pltpu.VMEM((CSEG,), jnp.float32),
          pltpu.VMEM((CSEG,), jnp.float32),
      ],
  )(dstp, e, den2)


def _aggregate_body(src_h, dst_h, cf_h, tab2_h,
                    h_out,
                    sseg, dseg, cseg, slist, dllist, cflist,
                    idxb, dlb, cfb, rows, acc, sem):
  c = lax.axis_index("c")
  s = lax.axis_index("s")
  iota = lax.iota(jnp.int32, L)
  coff = c * NPAD
  zv = jnp.zeros((L,), jnp.float32)

  def do_block(b, rem):
    for gs in range(4):
      sl = pl.ds(b * 64 + gs * L, L)
      jv = gs * L + iota
      mv = jv < rem
      idxb[pl.ds(gs * L, L)] = jnp.where(mv, slist[sl], 0) + coff
      dlb[pl.ds(gs * L, L)] = jnp.where(mv, dllist[sl], 0)
      cfb[pl.ds(gs * L, L)] = jnp.where(mv, cflist[sl], 0.0)
    pltpu.async_copy(tab2_h.at[idxb], rows, sem).wait()

    def accg(g, carry):
      dl16 = dlb[pl.ds(g * L, L)]
      cf16 = cfb[pl.ds(g * L, L)]
      for e in range(L):
        dl = dl16[e]
        cf = cf16[e]
        for v in range(8):
          acc[dl, pl.ds(v * L, L)] = (acc[dl, pl.ds(v * L, L)]
                                      + rows[g * L + e, pl.ds(v * L, L)] * cf)
      return carry

    lax.fori_loop(0, 4, accg, 0)

  for p in range(4):
    g = p * NS + s
    base = g * SLIV
    hi = jnp.where(g == 63, N + 48, base + SLIV) - base

    def zacc(i, carry):
      for v in range(8):
        acc[i, pl.ds(v * L, L)] = zv
      return carry

    lax.fori_loop(0, SLIV, zacc, 0)

    def seg_loop(sg, cnt):
      segoff = sg * SEG
      pltpu.sync_copy(src_h.at[pl.ds(segoff, SEG)], sseg)
      pltpu.sync_copy(dst_h.at[pl.ds(segoff, SEG)], dseg)
      pltpu.sync_copy(cf_h.at[pl.ds(segoff, SEG)], cseg)

      def comp(i, cnt2):
        d16 = dseg[pl.ds(i * L, L)]
        dl = d16 - base
        m = (dl >= 0) & (dl < hi)
        pos = plsc.cumsum(jnp.where(m, 1, 0))
        tgt = jnp.where(m, cnt2 + pos - 1, SEG + 127)
        plsc.store_scatter(slist, [tgt], sseg[pl.ds(i * L, L)], mask=m)
        plsc.store_scatter(dllist, [tgt], dl, mask=m)
        plsc.store_scatter(cflist, [tgt], cseg[pl.ds(i * L, L)], mask=m)
        return cnt2 + pos[L - 1]

      cnt = lax.fori_loop(0, SEGV, comp, cnt)
      nfull = cnt // 64

      def flush(b, carry):
        do_block(b, 64)
        return carry

      lax.fori_loop(0, nfull, flush, 0)
      rem = cnt - nfull * 64

      def mvrem(j, carry):
        slist[pl.ds(j * L, L)] = slist[pl.ds(nfull * 64 + j * L, L)]
        dllist[pl.ds(j * L, L)] = dllist[pl.ds(nfull * 64 + j * L, L)]
        cflist[pl.ds(j * L, L)] = cflist[pl.ds(nfull * 64 + j * L, L)]
        return carry

      lax.fori_loop(0, 4, mvrem, 0)
      return rem

    remf = lax.fori_loop(0, NSEG, seg_loop, 0)

    @pl.when(remf > 0)
    def _():
      do_block(0, remf)

    @pl.when(g == 63)
    def _():
      pltpu.sync_copy(acc.at[pl.ds(0, ND - 63 * SLIV)],
                      h_out.at[pl.ds(c * ND + 63 * SLIV, ND - 63 * SLIV)])

    @pl.when(g < 63)
    def _():
      pltpu.sync_copy(acc.at[pl.ds(0, SLIV)],
                      h_out.at[pl.ds(c * ND + g * SLIV, SLIV)])


def _sc_aggregate(srcp, dstp, cf, table):
  tab2 = jnp.concatenate([table[:, :128], table[:, 128:]], axis=0)
  return pl.kernel(
      _aggregate_body,
      out_type=jax.ShapeDtypeStruct((NC * ND, 128), jnp.float32),
      mesh=_mesh,
      compiler_params=_sc_params,
      scratch_types=[
          pltpu.VMEM((SEG,), jnp.int32),
          pltpu.VMEM((SEG,), jnp.int32),
          pltpu.VMEM((SEG,), jnp.float32),
          pltpu.VMEM((SEG + 128,), jnp.int32),
          pltpu.VMEM((SEG + 128,), jnp.int32),
          pltpu.VMEM((SEG + 128,), jnp.float32),
          pltpu.VMEM((64,), jnp.int32),
          pltpu.VMEM((64,), jnp.int32),
          pltpu.VMEM((64,), jnp.float32),
          pltpu.VMEM((64, 128), jnp.float32),
          pltpu.VMEM((SLIV, 128), jnp.float32),
          pltpu.SemaphoreType.DMA,
      ],
  )(srcp, dstp, cf, tab2)


def _readout_alpha_body(b_h, ss_h, td_h,
                        eg_out, dg_out,
                        bv, ssv, tdv, egv, bchunk, echunk, obuf, dg_sh):
  c = lax.axis_index("c")
  s = lax.axis_index("s")
  w = s * NC + c
  base = w * NPT

  def zfill(i, carry):
    obuf[pl.ds(i * L, L)] = jnp.zeros((L,), jnp.float32)
    return carry

  lax.fori_loop(0, GDT // L, zfill, 0)
  pltpu.sync_copy(obuf, dg_sh.at[pl.ds(s * GDT, GDT)])
  pltpu.sync_copy(b_h.at[pl.ds(base, NPT)], bv)
  pltpu.sync_copy(ss_h.at[pl.ds(base, NPT)], ssv)
  pltpu.sync_copy(td_h, tdv.at[pl.ds(0, G)])
  plsc.subcore_barrier()

  for i in range(NPT // L):
    b16 = bv[pl.ds(i * L, L)]
    al = _lrelu(ssv[pl.ds(i * L, L)] + plsc.load_gather(tdv, [b16]))
    egv[pl.ds(i * L, L)] = jnp.exp(al)

  pltpu.sync_copy(egv, eg_out.at[pl.ds(base, NPT)])

  def scat(k, carry):
    for j in range(5):
      bchunk[pl.ds(j * L, L)] = bv[pl.ds(k * 80 + j * L, L)]
      echunk[pl.ds(j * L, L)] = egv[pl.ds(k * 80 + j * L, L)]
    pltpu.sync_copy(echunk, dg_sh.at[bchunk], add=True)
    return carry

  lax.fori_loop(0, NPT // 80, scat, 0)
  plsc.subcore_barrier()
  pltpu.sync_copy(dg_sh.at[pl.ds(s * GDT, GDT)], obuf)
  pltpu.sync_copy(obuf, dg_out.at[pl.ds(c * GD + s * GDT, GDT)])


def _sc_readout_alpha(batchp, ss, td):
  return pl.kernel(
      _readout_alpha_body,
      out_type=(jax.ShapeDtypeStruct((NPAD,), jnp.float32),
                jax.ShapeDtypeStruct((NC * GD,), jnp.float32)),
      mesh=_mesh,
      compiler_params=_sc_params,
      scratch_types=[
          pltpu.VMEM((NPT,), jnp.int32),
          pltpu.VMEM((NPT,), jnp.float32),
          pltpu.VMEM((GD,), jnp.float32),
          pltpu.VMEM((NPT,), jnp.float32),
          pltpu.VMEM((80,), jnp.int32),
          pltpu.VMEM((80,), jnp.float32),
          pltpu.VMEM((GDT,), jnp.float32),
          pltpu.VMEM_SHARED((GD,), jnp.float32),
      ],
  )(batchp, ss, td)


def _readout_agg_body(b_h, eg_h, dg_h, tab_h,
                      g_out,
                      dg0v, dg1v, bb, egb, cb, rowsa, rowsb, zbuf,
                      gacca, gaccb, sem):
  c = lax.axis_index("c")
  s = lax.axis_index("s")
  w = s * NC + c
  base = w * NPT

  pltpu.sync_copy(dg_h.at[pl.ds(0, GD)], dg0v)
  pltpu.sync_copy(dg_h.at[pl.ds(GD, GD)], dg1v)
  zv = jnp.zeros((L,), jnp.float32)

  def zinit(i, carry):
    for v in range(8):
      zbuf[i, pl.ds(v * L, L)] = zv
    return carry

  lax.fori_loop(0, GDT, zinit, 0)
  pltpu.sync_copy(zbuf, gacca.at[pl.ds(s * GDT, GDT)])
  pltpu.sync_copy(zbuf, gaccb.at[pl.ds(s * GDT, GDT)])
  plsc.subcore_barrier()

  def blk(bi, carry):
    off = base + bi * B3
    pltpu.sync_copy(b_h.at[pl.ds(off, B3)], bb)
    pltpu.sync_copy(eg_h.at[pl.ds(off, B3)], egb)
    pltpu.sync_copy(tab_h.at[pl.ds(off, B3), pl.ds(0, 128)], rowsa)
    pltpu.sync_copy(tab_h.at[pl.ds(off, B3), pl.ds(128, 128)], rowsb)
    for i in range(B3 // L):
      b16 = bb[pl.ds(i * L, L)]
      den = (plsc.load_gather(dg0v, [b16]) + plsc.load_gather(dg1v, [b16])
             + 1e-16)
      cb[pl.ds(i * L, L)] = egb[pl.ds(i * L, L)] / den

    def scale(g, carry2):
      c16 = cb[pl.ds(g * L, L)]
      for e in range(L):
        r = g * L + e
        for v in range(8):
          rowsa[r, pl.ds(v * L, L)] = rowsa[r, pl.ds(v * L, L)] * c16[e]
          rowsb[r, pl.ds(v * L, L)] = rowsb[r, pl.ds(v * L, L)] * c16[e]
      return carry2

    lax.fori_loop(0, B3 // L, scale, 0)
    pltpu.sync_copy(rowsa, gacca.at[bb], add=True)
    pltpu.sync_copy(rowsb, gaccb.at[bb], add=True)
    return carry

  lax.fori_loop(0, NPT // B3, blk, 0)

  plsc.subcore_barrier()
  pltpu.sync_copy(gacca.at[pl.ds(s * GDT, GDT)], zbuf)
  pltpu.sync_copy(zbuf, g_out.at[pl.ds(c * GD + s * GDT, GDT), pl.ds(0, 128)])
  pltpu.sync_copy(gaccb.at[pl.ds(s * GDT, GDT)], zbuf)
  pltpu.sync_copy(zbuf,
                  g_out.at[pl.ds(c * GD + s * GDT, GDT), pl.ds(128, 128)])


def _sc_readout_agg(batchp, eg, dg2, table):
  return pl.kernel(
      _readout_agg_body,
      out_type=jax.ShapeDtypeStruct((NC * GD, HP), jnp.float32),
      mesh=_mesh,
      compiler_params=_sc_params,
      scratch_types=[
          pltpu.VMEM((GD,), jnp.float32),
          pltpu.VMEM((GD,), jnp.float32),
          pltpu.VMEM((B3,), jnp.int32),
          pltpu.VMEM((B3,), jnp.float32),
          pltpu.VMEM((B3,), jnp.float32),
          pltpu.VMEM((B3, 128), jnp.float32),
          pltpu.VMEM((B3, 128), jnp.float32),
          pltpu.VMEM((GDT, 128), jnp.float32),
          pltpu.VMEM_SHARED((GD, 128), jnp.float32),
          pltpu.VMEM_SHARED((GD, 128), jnp.float32),
          pltpu.SemaphoreType.DMA,
      ],
  )(batchp, eg, dg2, table)


BM = 400
NBLK = NPAD // BM
NBLK_REAL = N // BM


def _dot(a, b):
  return jax.lax.dot_general(a, b, (((1,), (0,)), ((), ())),
                             preferred_element_type=jnp.float32)


def _tc_main1_body(x_r, w1t_r, b1_r, wxt_r, w2t_r, attr_r,
                   x1_o, t1_o, r1_o, w1v_o):
  x1 = _lrelu(_dot(x_r[...], w1t_r[...]) + b1_r[...])
  x1_o[...] = x1
  t1_o[...] = _dot(x1, wxt_r[...])
  r1_o[...] = _dot(x1, attr_r[...])
  w1v_o[...] = _dot(x1, w2t_r[...])


def _tc_main1(x, w1t, b1, wxt, w2t, attr):
  return pl.pallas_call(
      _tc_main1_body,
      grid=(NBLK,),
      in_specs=[
          pl.BlockSpec((BM, 39), lambda i: (jnp.minimum(i, NBLK_REAL - 1), 0)),
          pl.BlockSpec((39, HP), lambda i: (0, 0)),
          pl.BlockSpec((1, HP), lambda i: (0, 0)),
          pl.BlockSpec((HP, HP), lambda i: (0, 0)),
          pl.BlockSpec((HP, HP), lambda i: (0, 0)),
          pl.BlockSpec((HP, 1), lambda i: (0, 0)),
      ],
      out_specs=[
          pl.BlockSpec((BM, HP), lambda i: (i, 0)),
          pl.BlockSpec((BM, HP), lambda i: (i, 0)),
          pl.BlockSpec((BM, 1), lambda i: (i, 0)),
          pl.BlockSpec((BM, HP), lambda i: (i, 0)),
      ],
      out_shape=[
          jax.ShapeDtypeStruct((NPAD, HP), jnp.float32),
          jax.ShapeDtypeStruct((NPAD, HP), jnp.float32),
          jax.ShapeDtypeStruct((NPAD, 1), jnp.float32),
          jax.ShapeDtypeStruct((NPAD, HP), jnp.float32),
      ],
  )(x, w1t, b1, wxt, w2t, attr)


BME = 1024


def _tc_e1_body(ea_r, wet_r, e1_o):
  e1_o[...] = _dot(ea_r[...], wet_r[...])


def _tc_e1(eap, wet):
  return pl.pallas_call(
      _tc_e1_body,
      grid=(EP // BME,),
      in_specs=[
          pl.BlockSpec((BME, 16), lambda i: (i, 0)),
          pl.BlockSpec((16, HP), lambda i: (0, 0)),
      ],
      out_specs=pl.BlockSpec((BME, HP), lambda i: (i, 0)),
      out_shape=jax.ShapeDtypeStruct((EP, HP), jnp.float32),
  )(eap, wet)


def _gru_block(h, xprev, wiht_r, whht_r, bih_r, bhh_r):
  gi0 = _dot(h, wiht_r[0]) + bih_r[0, 0]
  gi1 = _dot(h, wiht_r[1]) + bih_r[0, 1]
  gi2 = _dot(h, wiht_r[2]) + bih_r[0, 2]
  gh0 = _dot(xprev, whht_r[0]) + bhh_r[0, 0]
  gh1 = _dot(xprev, whht_r[1]) + bhh_r[0, 1]
  gh2 = _dot(xprev, whht_r[2]) + bhh_r[0, 2]
  r = jax.nn.sigmoid(gi0 + gh0)
  z = jax.nn.sigmoid(gi1 + gh1)
  n = jnp.tanh(gi2 + r * gh2)
  return (1.0 - z) * n + z * xprev


def _elu(v):
  return jnp.where(v > 0, v, jnp.exp(jnp.minimum(v, 0.0)) - 1.0)


def _tc_gru0_body(h0_r, h1_r, x1_r, gb_r, wiht_r, whht_r, bih_r, bhh_r,
                  gatwt_r, gas_r, gad_r,
                  x2_o, as_o, ad_o, xt_o):
  hs = jnp.concatenate([h0_r[...], h1_r[...]], axis=1)
  h = _elu(hs + gb_r[...])
  x2 = jax.nn.relu(_gru_block(h, x1_r[...], wiht_r, whht_r, bih_r, bhh_r))
  x2_o[...] = x2
  xt = _dot(x2, gatwt_r[...])
  as_o[...] = _dot(xt, gas_r[...])
  ad_o[...] = _dot(xt, gad_r[...])
  xt_o[...] = xt


def _tc_gru0(h0, h1, x1p, gb, wiht, whht, bih, bhh, gatwt, gas, gad):
  return pl.pallas_call(
      _tc_gru0_body,
      grid=(NBLK,),
      in_specs=[
          pl.BlockSpec((BM, 128), lambda i: (jnp.minimum(i, NBLK_REAL - 1), 0)),
          pl.BlockSpec((BM, 128), lambda i: (jnp.minimum(i, NBLK_REAL - 1), 0)),
          pl.BlockSpec((BM, HP), lambda i: (jnp.minimum(i, NBLK_REAL - 1), 0)),
          pl.BlockSpec((1, HP), lambda i: (0, 0)),
          pl.BlockSpec((3, HP, HP), lambda i: (0, 0, 0)),
          pl.BlockSpec((3, HP, HP), lambda i: (0, 0, 0)),
          pl.BlockSpec((1, 3, HP), lambda i: (0, 0, 0)),
          pl.BlockSpec((1, 3, HP), lambda i: (0, 0, 0)),
          pl.BlockSpec((HP, HP), lambda i: (0, 0)),
          pl.BlockSpec((HP, 1), lambda i: (0, 0)),
          pl.BlockSpec((HP, 1), lambda i: (0, 0)),
      ],
      out_specs=[
          pl.BlockSpec((BM, HP), lambda i: (i, 0)),
          pl.BlockSpec((BM, 1), lambda i: (i, 0)),
          pl.BlockSpec((BM, 1), lambda i: (i, 0)),
          pl.BlockSpec((BM, HP), lambda i: (i, 0)),
      ],
      out_shape=[
          jax.ShapeDtypeStruct((NPAD, HP), jnp.float32),
          jax.ShapeDtypeStruct((NPAD, 1), jnp.float32),
          jax.ShapeDtypeStruct((NPAD, 1), jnp.float32),
          jax.ShapeDtypeStruct((NPAD, HP), jnp.float32),
      ],
  )(h0, h1, x1p, gb, wiht, whht, bih, bhh, gatwt, gas, gad)


def _tc_gru1_body(h0_r, h1_r, x2_r, gb_r, wiht_r, whht_r, bih_r, bhh_r,
                  molwt_r, mas_r,
                  x3_o, xs_o, ss_o):
  hs = jnp.concatenate([h0_r[...], h1_r[...]], axis=1)
  h = _elu(hs + gb_r[...])
  x3 = jax.nn.relu(_gru_block(h, x2_r[...], wiht_r, whht_r, bih_r, bhh_r))
  x3_o[...] = x3
  xs = _dot(x3, molwt_r[...])
  xs_o[...] = xs
  ss_o[...] = _dot(xs, mas_r[...])


def _tc_gru1(h0, h1, x2p, gb, wiht, whht, bih, bhh, molwt, mas):
  return pl.pallas_call(
      _tc_gru1_body,
      grid=(NBLK,),
      in_specs=[
          pl.BlockSpec((BM, 128), lambda i: (jnp.minimum(i, NBLK_REAL - 1), 0)),
          pl.BlockSpec((BM, 128), lambda i: (jnp.minimum(i, NBLK_REAL - 1), 0)),
          pl.BlockSpec((BM, HP), lambda i: (jnp.minimum(i, NBLK_REAL - 1), 0)),
          pl.BlockSpec((1, HP), lambda i: (0, 0)),
          pl.BlockSpec((3, HP, HP), lambda i: (0, 0, 0)),
          pl.BlockSpec((3, HP, HP), lambda i: (0, 0, 0)),
          pl.BlockSpec((1, 3, HP), lambda i: (0, 0, 0)),
          pl.BlockSpec((1, 3, HP), lambda i: (0, 0, 0)),
          pl.BlockSpec((HP, HP), lambda i: (0, 0)),
          pl.BlockSpec((HP, 1), lambda i: (0, 0)),
      ],
      out_specs=[
          pl.BlockSpec((BM, HP), lambda i: (i, 0)),
          pl.BlockSpec((BM, HP), lambda i: (i, 0)),
          pl.BlockSpec((BM, 1), lambda i: (i, 0)),
      ],
      out_shape=[
          jax.ShapeDtypeStruct((NPAD, HP), jnp.float32),
          jax.ShapeDtypeStruct((NPAD, HP), jnp.float32),
          jax.ShapeDtypeStruct((NPAD, 1), jnp.float32),
      ],
  )(h0, h1, x2p, gb, wiht, whht, bih, bhh, molwt, mas)


def _tc_pool_body(p_r, molwt_r, mad_r, out_o, td_o):
  out0 = jax.nn.relu(p_r[0, :G, :] + p_r[1, :G, :])
  out_o[...] = out0
  td_o[...] = _dot(_dot(out0, molwt_r[...]), mad_r[...])


def _tc_pool(pool2, molwt, mad):
  return pl.pallas_call(
      _tc_pool_body,
      grid=(1,),
      in_specs=[
          pl.BlockSpec((NC, GD, HP), lambda i: (0, 0, 0)),
          pl.BlockSpec((HP, HP), lambda i: (0, 0)),
          pl.BlockSpec((HP, 1), lambda i: (0, 0)),
      ],
      out_specs=[
          pl.BlockSpec((G, HP), lambda i: (0, 0)),
          pl.BlockSpec((G, 1), lambda i: (0, 0)),
      ],
      out_shape=[
          jax.ShapeDtypeStruct((G, HP), jnp.float32),
          jax.ShapeDtypeStruct((G, 1), jnp.float32),
      ],
  )(pool2, molwt, mad)


def _tc_mgru_body(g_r, op_r, mb_r, wiht_r, whht_r, bih_r, bhh_r,
                  molwt_r, mad_r, l2t_r, l2b_r,
                  out_o, td_o, y_o):
  h = _elu(g_r[0, :G, :] + g_r[1, :G, :] + mb_r[...])
  outn = jax.nn.relu(_gru_block(h, op_r[...], wiht_r, whht_r, bih_r, bhh_r))
  out_o[...] = outn
  td_o[...] = _dot(_dot(outn, molwt_r[...]), mad_r[...])
  y_o[...] = _dot(outn, l2t_r[...]) + l2b_r[...]


def _tc_mgru(gs, outp, mb, wiht, whht, bih, bhh, molwt, mad, l2t, l2b):
  return pl.pallas_call(
      _tc_mgru_body,
      grid=(1,),
      in_specs=[
          pl.BlockSpec((NC, GD, HP), lambda i: (0, 0, 0)),
          pl.BlockSpec((G, HP), lambda i: (0, 0)),
          pl.BlockSpec((1, HP), lambda i: (0, 0)),
          pl.BlockSpec((3, HP, HP), lambda i: (0, 0, 0)),
          pl.BlockSpec((3, HP, HP), lambda i: (0, 0, 0)),
          pl.BlockSpec((1, 3, HP), lambda i: (0, 0, 0)),
          pl.BlockSpec((1, 3, HP), lambda i: (0, 0, 0)),
          pl.BlockSpec((HP, HP), lambda i: (0, 0)),
          pl.BlockSpec((HP, 1), lambda i: (0, 0)),
          pl.BlockSpec((HP, 1), lambda i: (0, 0)),
          pl.BlockSpec((1, 1), lambda i: (0, 0)),
      ],
      out_specs=[
          pl.BlockSpec((G, HP), lambda i: (0, 0)),
          pl.BlockSpec((G, 1), lambda i: (0, 0)),
          pl.BlockSpec((G, 1), lambda i: (0, 0)),
      ],
      out_shape=[
          jax.ShapeDtypeStruct((G, HP), jnp.float32),
          jax.ShapeDtypeStruct((G, 1), jnp.float32),
          jax.ShapeDtypeStruct((G, 1), jnp.float32),
      ],
  )(gs, outp, mb, wiht, whht, bih, bhh, molwt, mad, l2t, l2b)


def _padw(w, rows, cols):
  return jnp.pad(w, ((0, rows - w.shape[0]), (0, cols - w.shape[1])))


def _pack_gru(p, pre):
  wih = p[pre + "_Wih"]
  whh = p[pre + "_Whh"]
  bih = p[pre + "_bih"]
  bhh = p[pre + "_bhh"]
  wiht = jnp.stack([_padw(wih[k * H:(k + 1) * H].T, HP, HP) for k in range(3)])
  whht = jnp.stack([_padw(whh[k * H:(k + 1) * H].T, HP, HP) for k in range(3)])
  biht = jnp.stack([jnp.pad(bih[k * H:(k + 1) * H], (0, HP - H))
                    for k in range(3)])[None]
  bhht = jnp.stack([jnp.pad(bhh[k * H:(k + 1) * H], (0, HP - H))
                    for k in range(3)])[None]
  return wiht, whht, biht, bhht


@jax.jit
def _run(x, edge_index, edge_attr, batch, p):
  src, dst = edge_index[0], edge_index[1]
  padn = EP - E
  srcp = jnp.concatenate([src, (jnp.arange(padn, dtype=jnp.int32) * 61) % N])
  dstp = jnp.concatenate([dst, jnp.full((padn,), N, jnp.int32)])
  batchp = jnp.concatenate(
      [batch, jnp.full((NPAD - N,), G, jnp.int32)]).astype(jnp.int32)
  eap = jnp.pad(edge_attr, ((0, EP - E), (0, 16 - 10)))
  ones_np = jnp.ones((NPAD,), jnp.float32)
  den_one = jnp.concatenate([jnp.ones((GD,), jnp.float32),
                             jnp.zeros((GD,), jnp.float32)])

  w1t = jnp.pad(p["lin1_W"].T, ((0, 0), (0, HP - H)))
  b1 = jnp.pad(p["lin1_b"], (0, HP - H))[None]
  wxt = _padw(p["gate_lin1_W"][:, :H].T, HP, HP)
  wet = jnp.pad(p["gate_lin1_W"][:, H:].T, ((0, 16 - 10), (0, HP - H)))
  w2t = _padw(p["gate_lin2_W"].T, HP, HP)
  attl = jnp.pad(p["gate_att_l"], (0, HP - H))
  attr = jnp.pad(p["gate_att_r"], (0, HP - H))[:, None]
  gb = jnp.pad(p["gate_bias"], (0, HP - H))[None]
  gatwt = _padw(p["gat_W"].T, HP, HP)
  gas = jnp.pad(p["gat_att_src"], (0, HP - H))[:, None]
  gad = jnp.pad(p["gat_att_dst"], (0, HP - H))[:, None]
  gatb = jnp.pad(p["gat_bias"], (0, HP - H))[None]
  molwt = _padw(p["mol_W"].T, HP, HP)
  mas = jnp.pad(p["mol_att_src"], (0, HP - H))[:, None]
  mad = jnp.pad(p["mol_att_dst"], (0, HP - H))[:, None]
  molb = jnp.pad(p["mol_bias"], (0, HP - H))[None]
  l2t = jnp.pad(p["lin2_W"].T, ((0, HP - H), (0, 0)))
  l2b = p["lin2_b"][None]
  g0 = _pack_gru(p, "gru0")
  g1 = _pack_gru(p, "gru1")
  mg = _pack_gru(p, "mgru")

  x1p, t1, r1, w1v = _tc_main1(x, w1t, b1, wxt, w2t, attr)
  e1 = _tc_e1(eap, wet)
  ev, den2 = _sc_alpha_gate(srcp, dstp, t1, e1, r1.reshape(NPAD), attl)
  cf1 = _sc_coeff(dstp, ev, den2)
  h1f = _sc_aggregate(srcp, dstp, cf1, w1v)
  x2p, asr, adr, xtp = _tc_gru0(h1f[:ND], h1f[ND:], x1p, gb, *g0,
                                gatwt, gas, gad)
  ev2, den2b = _sc_alpha_gat(srcp, dstp, asr.reshape(NPAD),
                             adr.reshape(NPAD))
  cf2 = _sc_coeff(dstp, ev2, den2b)
  h2f = _sc_aggregate(srcp, dstp, cf2, xtp)
  x3p, xsp, ssr = _tc_gru1(h2f[:ND], h2f[ND:], x2p, gatb, *g1, molwt, mas)
  pool2 = _sc_readout_agg(batchp, ones_np, den_one, x3p).reshape(NC, GD, HP)
  outp, td = _tc_pool(pool2, molwt, mad)
  y = None
  for _ in range(2):
    eg, dg2 = _sc_readout_alpha(batchp, ssr.reshape(NPAD), td.reshape(G))
    gs = _sc_readout_agg(batchp, eg, dg2, xsp).reshape(NC, GD, HP)
    outp, td, y = _tc_mgru(gs, outp, molb, *mg, molwt, mad, l2t, l2b)
  return y


def kernel(x, edge_index, edge_attr, batch, params):
  return _run(x, edge_index, edge_attr, batch, params)
